# Initial kernel scaffold; baseline (speedup 1.0000x reference)
import jax
import jax.numpy as jnp
from jax.experimental import pallas as pl


def kernel(inputs, teacher_states, edge_index, batch_cnt, enc_W, enc_b, dec_W, dec_b, W_out, b_out):
    raise NotImplementedError("write your pallas kernel here")


# speedup vs baseline: 2.7232x; 2.7232x over previous
import functools

import jax
import jax.numpy as jnp
from jax import lax
from jax.experimental import pallas as pl
from jax.experimental.pallas import tpu as pltpu
from jax.experimental.pallas import tpu_sc as plsc

N = 10000
F = 128
NPAD = 10240
DEAD = 10100
NW = 32
CH = 128
ZR = NPAD // 16


def _make_spmm(n_chunks):
    mesh = plsc.VectorSubcoreMesh(core_axis_name="c", subcore_axis_name="s")

    @functools.partial(
        pl.kernel,
        mesh=mesh,
        out_type=jax.ShapeDtypeStruct((2, NPAD, F), jnp.float32),
        scratch_types=[
            pltpu.VMEM((n_chunks, CH), jnp.int32),
            pltpu.VMEM((n_chunks, CH), jnp.int32),
            pltpu.VMEM((CH, F), jnp.float32),
            pltpu.VMEM((128, F), jnp.float32),
            pltpu.VMEM_SHARED((NPAD, F), jnp.float32),
            pltpu.SemaphoreType.DMA,
        ],
    )
    def spmm(table, src_w, dst_w, out, src_v, dst_v, rows_v, zbuf, acc, sem):
        c = lax.axis_index("c")
        s = lax.axis_index("s")
        w = s * 2 + c

        pltpu.sync_copy(src_w.at[w], src_v)
        pltpu.sync_copy(dst_w.at[w], dst_v)

        zvec = jnp.zeros((16,), jnp.float32)

        def zrow(i, _):
            for j in range(F // 16):
                zbuf[i, pl.ds(j * 16, 16)] = zvec
            return 0

        lax.fori_loop(0, 128, zrow, 0)
        for j in range(ZR // 128):
            pltpu.sync_copy(zbuf, acc.at[pl.ds(s * ZR + j * 128, 128)])
        plsc.subcore_barrier()

        def chunk(k, _):
            pltpu.async_copy(table.at[src_v.at[k]], rows_v, sem).wait()
            pltpu.sync_copy(rows_v, acc.at[dst_v.at[k]], add=True)
            return 0

        lax.fori_loop(0, n_chunks, chunk, 0)
        plsc.subcore_barrier()

        pltpu.sync_copy(
            acc.at[pl.ds(s * ZR, ZR)],
            out.at[c].at[pl.ds(s * ZR, ZR)],
        )

    return spmm


_R = 1000
_GRID = N // _R


def _node_spec(shape_prefix=()):
    nd = len(shape_prefix)
    return pl.BlockSpec(shape_prefix + (_R, F),
                        lambda i, _nd=nd: (0,) * _nd + (i, 0))


def _full_spec(shape):
    nd = len(shape)
    return pl.BlockSpec(shape, lambda i, _nd=nd: (0,) * _nd)


def _stage1_call(gx, gh, h, degcol, Wx, Wh, b, proj=None):
    has_proj = proj is not None

    def body(*refs):
        if has_proj:
            (gx_r, gh_r, h_r, deg_r, wx_r, wh_r, b_r, wo_r, bo_r,
             u_o, rh_o, cx_o) = refs
        else:
            (gx_r, gh_r, h_r, deg_r, wx_r, wh_r, b_r,
             u_o, rh_o, cx_o) = refs
        deg = jnp.maximum(deg_r[...], 1.0)
        invd = 1.0 / deg
        axn = (gx_r[0] + gx_r[1]) * invd
        if has_proj:
            ones_msk = jnp.minimum(deg_r[...], 1.0)
            axn = jnp.dot(axn, wo_r[...],
                          preferred_element_type=jnp.float32) + ones_msk * bo_r[...]
        ahn = (gh_r[0] + gh_r[1]) * invd
        pre_r = (jnp.dot(axn, wx_r[0], preferred_element_type=jnp.float32)
                 + jnp.dot(ahn, wh_r[0], preferred_element_type=jnp.float32)
                 + b_r[0])
        pre_u = (jnp.dot(axn, wx_r[1], preferred_element_type=jnp.float32)
                 + jnp.dot(ahn, wh_r[1], preferred_element_type=jnp.float32)
                 + b_r[1])
        r = jax.nn.sigmoid(pre_r)
        u_o[...] = jax.nn.sigmoid(pre_u)
        rh_o[...] = r * h_r[...]
        cx_o[...] = (jnp.dot(axn, wx_r[2], preferred_element_type=jnp.float32)
                     + b_r[2])

    in_specs = [
        _node_spec((2,)), _node_spec((2,)), _node_spec(),
        pl.BlockSpec((_R, 1), lambda i: (i, 0)),
        _full_spec((3, F, F)), _full_spec((3, F, F)), _full_spec((3, 1, F)),
    ]
    args = [gx, gh, h, degcol, Wx, Wh, b.reshape(3, 1, F)]
    if has_proj:
        in_specs += [_full_spec((F, F)), _full_spec((1, F))]
        args += [proj[0], proj[1].reshape(1, F)]
    out_shape = [jax.ShapeDtypeStruct((N, F), jnp.float32)] * 3
    return pl.pallas_call(
        body,
        grid=(_GRID,),
        in_specs=in_specs,
        out_specs=[_node_spec()] * 3,
        out_shape=out_shape,
    )(*args)


def _stage2_call(u, h, cx, garh, Wch, degcol, proj=None):
    has_proj = proj is not None

    def body(*refs):
        if has_proj:
            u_r, h_r, cx_r, garh_r, wch_r, deg_r, wo_r, bo_r, h_o, x_o = refs
        else:
            u_r, h_r, cx_r, garh_r, wch_r, deg_r, h_o = refs
        invd = 1.0 / jnp.maximum(deg_r[...], 1.0)
        arhn = (garh_r[0] + garh_r[1]) * invd
        cand = jnp.tanh(cx_r[...] + jnp.dot(arhn, wch_r[...],
                                            preferred_element_type=jnp.float32))
        u = u_r[...]
        hn = u * h_r[...] + (1.0 - u) * cand
        h_o[...] = hn
        if has_proj:
            x_o[...] = jnp.dot(hn, wo_r[...],
                               preferred_element_type=jnp.float32) + bo_r[...]

    in_specs = [
        _node_spec(), _node_spec(), _node_spec(), _node_spec((2,)),
        _full_spec((F, F)),
        pl.BlockSpec((_R, 1), lambda i: (i, 0)),
    ]
    args = [u, h, cx, garh, Wch, degcol]
    out_specs = [_node_spec()]
    out_shape = [jax.ShapeDtypeStruct((N, F), jnp.float32)]
    if has_proj:
        in_specs += [_full_spec((F, F)), _full_spec((1, F))]
        args += [proj[0], proj[1].reshape(1, F)]
        out_specs.append(_node_spec())
        out_shape.append(jax.ShapeDtypeStruct((N, F), jnp.float32))
    res = pl.pallas_call(
        body,
        grid=(_GRID,),
        in_specs=in_specs,
        out_specs=out_specs,
        out_shape=out_shape,
    )(*args)
    return res if has_proj else res[0]


def kernel(inputs, teacher_states, edge_index, batch_cnt,
           enc_W, enc_b, dec_W, dec_b, W_out, b_out):
    t_len = inputs.shape[0]
    src = edge_index[0]
    dst = edge_index[1]
    e = src.shape[0]

    per_w = e // NW
    per_w_pad = -(-per_w // CH) * CH
    n_chunks = per_w_pad // CH
    pad = per_w_pad - per_w
    src_w = jnp.concatenate(
        [src.reshape(NW, per_w), jnp.zeros((NW, pad), jnp.int32)], axis=1
    ).reshape(NW, n_chunks, CH)
    dst_w = jnp.concatenate(
        [dst.reshape(NW, per_w), jnp.full((NW, pad), DEAD, jnp.int32)], axis=1
    ).reshape(NW, n_chunks, CH)

    spmm = _make_spmm(n_chunks)

    def agg(table):
        return spmm(table, src_w, dst_w)

    degp = agg(jnp.ones((N, F), jnp.float32))
    degcol = degp[0, :, :1] + degp[1, :, :1]

    zeros_p = jnp.zeros((2, NPAD, F), jnp.float32)
    h0 = jnp.zeros((N, F), jnp.float32)
    h1 = jnp.zeros((N, F), jnp.float32)
    g0 = zeros_p
    g1 = zeros_p

    enc_Wx = enc_W[:, :, :F, :]
    enc_Wh = enc_W[:, :, F:, :]
    dec_Wx = dec_W[:, :, :F, :]
    dec_Wh = dec_W[:, :, F:, :]

    for t in range(t_len):
        ax = agg(inputs[t])
        u, rh, cx = _stage1_call(ax, g0, h0, degcol, enc_Wx[0], enc_Wh[0], enc_b[0])
        arh = agg(rh) if t > 0 else zeros_p
        h0 = _stage2_call(u, h0, cx, arh, enc_Wh[0, 2], degcol)
        g0 = agg(h0)
        u, rh, cx = _stage1_call(g0, g1, h1, degcol, enc_Wx[1], enc_Wh[1], enc_b[1])
        arh = agg(rh) if t > 0 else zeros_p
        h1 = _stage2_call(u, h1, cx, arh, enc_Wh[1, 2], degcol)
        g1 = agg(h1)

    outs = []
    for t in range(t_len):
        if t == 0:
            u, rh, cx = _stage1_call(zeros_p, g0, h0, degcol,
                                     dec_Wx[0], dec_Wh[0], dec_b[0])
        else:
            u, rh, cx = _stage1_call(g1, g0, h0, degcol,
                                     dec_Wx[0], dec_Wh[0], dec_b[0],
                                     proj=(W_out, b_out))
        arh = agg(rh)
        h0 = _stage2_call(u, h0, cx, arh, dec_Wh[0, 2], degcol)
        g0 = agg(h0)
        u, rh, cx = _stage1_call(g0, g1, h1, degcol, dec_Wx[1], dec_Wh[1], dec_b[1])
        arh = agg(rh)
        h1, x = _stage2_call(u, h1, cx, arh, dec_Wh[1, 2], degcol,
                             proj=(W_out, b_out))
        if t < t_len - 1:
            g1 = agg(h1)
        outs.append(x)
    return jnp.stack(outs)


# speedup vs baseline: 2.7249x; 1.0006x over previous
import functools

import jax
import jax.numpy as jnp
from jax import lax
from jax.experimental import pallas as pl
from jax.experimental.pallas import tpu as pltpu
from jax.experimental.pallas import tpu_sc as plsc

N = 10000
F = 128
NPAD = 10240
DEAD = 10100
NW = 32
CH = 128
ZR = NPAD // 16


def _make_spmm(n_chunks):
    mesh = plsc.VectorSubcoreMesh(core_axis_name="c", subcore_axis_name="s")

    @functools.partial(
        pl.kernel,
        mesh=mesh,
        out_type=jax.ShapeDtypeStruct((2, NPAD, F), jnp.float32),
        scratch_types=[
            pltpu.VMEM((n_chunks, CH), jnp.int32),
            pltpu.VMEM((n_chunks, CH), jnp.int32),
            pltpu.VMEM((CH, F), jnp.float32),
            pltpu.VMEM((CH, F), jnp.float32),
            pltpu.VMEM((128, F), jnp.float32),
            pltpu.VMEM_SHARED((NPAD, F), jnp.float32),
            pltpu.SemaphoreType.DMA,
            pltpu.SemaphoreType.DMA,
        ],
    )
    def spmm(table, src_w, dst_w, out,
             src_v, dst_v, rows_a, rows_b, zbuf, acc, sem_a, sem_b):
        c = lax.axis_index("c")
        s = lax.axis_index("s")
        w = s * 2 + c

        pltpu.sync_copy(src_w.at[w], src_v)
        pltpu.sync_copy(dst_w.at[w], dst_v)

        zvec = jnp.zeros((16,), jnp.float32)

        def zrow(i, _):
            for j in range(F // 16):
                zbuf[i, pl.ds(j * 16, 16)] = zvec
            return 0

        lax.fori_loop(0, 128, zrow, 0)
        for j in range(ZR // 128):
            pltpu.sync_copy(zbuf, acc.at[pl.ds(s * ZR + j * 128, 128)])
        plsc.subcore_barrier()

        def chunk(k, _):
            pltpu.async_copy(table.at[src_v.at[k]], rows_a, sem_a).wait()
            pltpu.sync_copy(rows_a, acc.at[dst_v.at[k]], add=True)
            return 0

        lax.fori_loop(0, n_chunks, chunk, 0)
        plsc.subcore_barrier()

        pltpu.sync_copy(
            acc.at[pl.ds(s * ZR, ZR)],
            out.at[c].at[pl.ds(s * ZR, ZR)],
        )

    return spmm


_R = 1000
_GRID = N // _R


def _node_spec(shape_prefix=()):
    nd = len(shape_prefix)
    return pl.BlockSpec(shape_prefix + (_R, F),
                        lambda i, _nd=nd: (0,) * _nd + (i, 0))


def _full_spec(shape):
    nd = len(shape)
    return pl.BlockSpec(shape, lambda i, _nd=nd: (0,) * _nd)


def _stage1_call(gx, gh, h, degcol, Wx, Wh, b, proj=None):
    has_proj = proj is not None

    def body(*refs):
        if has_proj:
            (gx_r, gh_r, h_r, deg_r, wx_r, wh_r, b_r, wo_r, bo_r,
             u_o, rh_o, cx_o) = refs
        else:
            (gx_r, gh_r, h_r, deg_r, wx_r, wh_r, b_r,
             u_o, rh_o, cx_o) = refs
        deg = jnp.maximum(deg_r[...], 1.0)
        invd = 1.0 / deg
        axn = (gx_r[0] + gx_r[1]) * invd
        if has_proj:
            ones_msk = jnp.minimum(deg_r[...], 1.0)
            axn = jnp.dot(axn, wo_r[...],
                          preferred_element_type=jnp.float32) + ones_msk * bo_r[...]
        ahn = (gh_r[0] + gh_r[1]) * invd
        pre_r = (jnp.dot(axn, wx_r[0], preferred_element_type=jnp.float32)
                 + jnp.dot(ahn, wh_r[0], preferred_element_type=jnp.float32)
                 + b_r[0])
        pre_u = (jnp.dot(axn, wx_r[1], preferred_element_type=jnp.float32)
                 + jnp.dot(ahn, wh_r[1], preferred_element_type=jnp.float32)
                 + b_r[1])
        r = jax.nn.sigmoid(pre_r)
        u_o[...] = jax.nn.sigmoid(pre_u)
        rh_o[...] = r * h_r[...]
        cx_o[...] = (jnp.dot(axn, wx_r[2], preferred_element_type=jnp.float32)
                     + b_r[2])

    in_specs = [
        _node_spec((2,)), _node_spec((2,)), _node_spec(),
        pl.BlockSpec((_R, 1), lambda i: (i, 0)),
        _full_spec((3, F, F)), _full_spec((3, F, F)), _full_spec((3, 1, F)),
    ]
    args = [gx, gh, h, degcol, Wx, Wh, b.reshape(3, 1, F)]
    if has_proj:
        in_specs += [_full_spec((F, F)), _full_spec((1, F))]
        args += [proj[0], proj[1].reshape(1, F)]
    out_shape = [jax.ShapeDtypeStruct((N, F), jnp.float32)] * 3
    return pl.pallas_call(
        body,
        grid=(_GRID,),
        in_specs=in_specs,
        out_specs=[_node_spec()] * 3,
        out_shape=out_shape,
    )(*args)


def _stage2_call(u, h, cx, garh, Wch, degcol, proj=None):
    has_proj = proj is not None

    def body(*refs):
        if has_proj:
            u_r, h_r, cx_r, garh_r, wch_r, deg_r, wo_r, bo_r, h_o, x_o = refs
        else:
            u_r, h_r, cx_r, garh_r, wch_r, deg_r, h_o = refs
        invd = 1.0 / jnp.maximum(deg_r[...], 1.0)
        arhn = (garh_r[0] + garh_r[1]) * invd
        cand = jnp.tanh(cx_r[...] + jnp.dot(arhn, wch_r[...],
                                            preferred_element_type=jnp.float32))
        u = u_r[...]
        hn = u * h_r[...] + (1.0 - u) * cand
        h_o[...] = hn
        if has_proj:
            x_o[...] = jnp.dot(hn, wo_r[...],
                               preferred_element_type=jnp.float32) + bo_r[...]

    in_specs = [
        _node_spec(), _node_spec(), _node_spec(), _node_spec((2,)),
        _full_spec((F, F)),
        pl.BlockSpec((_R, 1), lambda i: (i, 0)),
    ]
    args = [u, h, cx, garh, Wch, degcol]
    out_specs = [_node_spec()]
    out_shape = [jax.ShapeDtypeStruct((N, F), jnp.float32)]
    if has_proj:
        in_specs += [_full_spec((F, F)), _full_spec((1, F))]
        args += [proj[0], proj[1].reshape(1, F)]
        out_specs.append(_node_spec())
        out_shape.append(jax.ShapeDtypeStruct((N, F), jnp.float32))
    res = pl.pallas_call(
        body,
        grid=(_GRID,),
        in_specs=in_specs,
        out_specs=out_specs,
        out_shape=out_shape,
    )(*args)
    return res if has_proj else res[0]


def kernel(inputs, teacher_states, edge_index, batch_cnt,
           enc_W, enc_b, dec_W, dec_b, W_out, b_out):
    t_len = inputs.shape[0]
    src = edge_index[0]
    dst = edge_index[1]
    e = src.shape[0]

    per_w = e // NW
    per_w_pad = -(-per_w // CH) * CH
    n_chunks = per_w_pad // CH
    pad = per_w_pad - per_w
    src_w = jnp.concatenate(
        [src.reshape(NW, per_w), jnp.zeros((NW, pad), jnp.int32)], axis=1
    ).reshape(NW, n_chunks, CH)
    dst_w = jnp.concatenate(
        [dst.reshape(NW, per_w), jnp.full((NW, pad), DEAD, jnp.int32)], axis=1
    ).reshape(NW, n_chunks, CH)

    spmm = _make_spmm(n_chunks)

    def agg(table):
        return spmm(table, src_w, dst_w)

    degp = agg(jnp.ones((N, F), jnp.float32))
    degcol = degp[0, :, :1] + degp[1, :, :1]

    zeros_p = jnp.zeros((2, NPAD, F), jnp.float32)
    h0 = jnp.zeros((N, F), jnp.float32)
    h1 = jnp.zeros((N, F), jnp.float32)
    g0 = zeros_p
    g1 = zeros_p

    enc_Wx = enc_W[:, :, :F, :]
    enc_Wh = enc_W[:, :, F:, :]
    dec_Wx = dec_W[:, :, :F, :]
    dec_Wh = dec_W[:, :, F:, :]

    for t in range(t_len):
        ax = agg(inputs[t])
        u, rh, cx = _stage1_call(ax, g0, h0, degcol, enc_Wx[0], enc_Wh[0], enc_b[0])
        arh = agg(rh) if t > 0 else zeros_p
        h0 = _stage2_call(u, h0, cx, arh, enc_Wh[0, 2], degcol)
        g0 = agg(h0)
        u, rh, cx = _stage1_call(g0, g1, h1, degcol, enc_Wx[1], enc_Wh[1], enc_b[1])
        arh = agg(rh) if t > 0 else zeros_p
        h1 = _stage2_call(u, h1, cx, arh, enc_Wh[1, 2], degcol)
        g1 = agg(h1)

    outs = []
    for t in range(t_len):
        if t == 0:
            u, rh, cx = _stage1_call(zeros_p, g0, h0, degcol,
                                     dec_Wx[0], dec_Wh[0], dec_b[0])
        else:
            u, rh, cx = _stage1_call(g1, g0, h0, degcol,
                                     dec_Wx[0], dec_Wh[0], dec_b[0],
                                     proj=(W_out, b_out))
        arh = agg(rh)
        h0 = _stage2_call(u, h0, cx, arh, dec_Wh[0, 2], degcol)
        g0 = agg(h0)
        u, rh, cx = _stage1_call(g0, g1, h1, degcol, dec_Wx[1], dec_Wh[1], dec_b[1])
        arh = agg(rh)
        h1, x = _stage2_call(u, h1, cx, arh, dec_Wh[1, 2], degcol,
                             proj=(W_out, b_out))
        if t < t_len - 1:
            g1 = agg(h1)
        outs.append(x)
    return jnp.stack(outs)
